# baseline (speedup 1.0000x reference)
import functools

import numpy as np
import jax
import jax.numpy as jnp
from jax.experimental import pallas as pl
from jax.experimental.pallas import tpu as pltpu

PROB = 0.25
LH = 8


@functools.lru_cache(maxsize=None)
def _drop_indices(b: int):
    cpu = jax.devices("cpu")[0]
    with jax.default_device(cpu):
        perm = np.asarray(jax.random.permutation(jax.random.key(1), b))
    n = 1 if b == 1 else int(b * PROB)
    return tuple(int(i) for i in perm[:n])


def _prev_kept_table(b, drop):
    tab, prev = [], None
    for i in range(b):
        if i not in drop:
            prev = i
        tab.append(prev)
    first_kept = next(i for i in range(b) if i not in drop)
    return tuple(first_kept if v is None else v for v in tab)


def _prev_drop_table(b, drop):
    tab, prev = [], None
    for i in range(b):
        if i in drop:
            prev = i
        tab.append(prev)
    first_drop = min(drop)
    return tuple(first_drop if v is None else v for v in tab)


def _masked_copy_kernel(x_hbm, last_ref, o_ref, sem, *, drop, t):
    i = pl.program_id(0)
    dropped = functools.reduce(jnp.logical_or, [i == di for di in drop])

    @pl.when(~dropped)
    def _copy():
        dma = pltpu.make_async_copy(x_hbm.at[pl.ds(i, 1)], o_ref, sem)
        dma.start()
        dma.wait()

    @pl.when(dropped)
    def _zero():
        last = last_ref[0, LH - 1, :, :]
        tids = jax.lax.broadcasted_iota(jnp.int32, o_ref.shape, 1)
        o_ref[...] = jnp.where(tids == t - 1, last[None, None], 0.0)


@functools.partial(jax.jit, static_argnums=(1,))
def _run(emb0, drop):
    b, t, c, d = emb0.shape
    prev_kept = _prev_kept_table(b, drop)
    prev_drop = _prev_drop_table(b, drop)

    def in_map(i):
        p = i
        for di in drop:
            p = jnp.where(i == di, prev_kept[di], p)
        return (p, 0, 0, 0)

    def last_map(i):
        p = prev_drop[0]
        for idx in range(1, b):
            if prev_drop[idx] != prev_drop[idx - 1]:
                p = jnp.where(i >= idx, prev_drop[idx], p)
        return (p, t // LH - 1, 0, 0)

    return pl.pallas_call(
        functools.partial(_masked_copy_kernel, drop=drop, t=t),
        grid=(b,),
        in_specs=[
            pl.BlockSpec(memory_space=pl.ANY),
            pl.BlockSpec((1, LH, c, d), last_map),
        ],
        out_specs=pl.BlockSpec((1, t, c, d), lambda i: (i, 0, 0, 0)),
        out_shape=jax.ShapeDtypeStruct((b, t, c, d), emb0.dtype),
        scratch_shapes=[pltpu.SemaphoreType.DMA],
        compiler_params=pltpu.CompilerParams(
            dimension_semantics=("arbitrary",)),
    )(emb0, emb0)


_drop_indices(16)


def kernel(emb0):
    return _run(emb0, _drop_indices(emb0.shape[0]))


# speedup vs baseline: 1.3247x; 1.3247x over previous
import functools

import numpy as np
import jax
import jax.numpy as jnp
from jax.experimental import pallas as pl
from jax.experimental.pallas import tpu as pltpu

PROB = 0.25
LH = 8


@functools.lru_cache(maxsize=None)
def _drop_indices(b: int):
    cpu = jax.devices("cpu")[0]
    with jax.default_device(cpu):
        perm = np.asarray(jax.random.permutation(jax.random.key(1), b))
    n = 1 if b == 1 else int(b * PROB)
    return tuple(int(i) for i in perm[:n])


def _prev_kept_table(b, drop):
    tab, prev = [], None
    for i in range(b):
        if i not in drop:
            prev = i
        tab.append(prev)
    first_kept = next(i for i in range(b) if i not in drop)
    return tuple(first_kept if v is None else v for v in tab)


def _prev_drop_table(b, drop):
    tab, prev = [], None
    for i in range(b):
        if i in drop:
            prev = i
        tab.append(prev)
    first_drop = min(drop)
    return tuple(first_drop if v is None else v for v in tab)


def _masked_copy_kernel(x_ref, last_ref, o_ref, *, drop, t):
    i = pl.program_id(0)
    dropped = functools.reduce(jnp.logical_or, [i == di for di in drop])

    @pl.when(~dropped)
    def _copy():
        o_ref[...] = x_ref[...]

    @pl.when(dropped)
    def _zero():
        last = last_ref[0, LH - 1, :, :]
        tids = jax.lax.broadcasted_iota(jnp.int32, o_ref.shape, 1)
        o_ref[...] = jnp.where(tids == t - 1, last[None, None], 0.0)


@functools.partial(jax.jit, static_argnums=(1,))
def _run(emb0, drop):
    b, t, c, d = emb0.shape
    prev_kept = _prev_kept_table(b, drop)
    prev_drop = _prev_drop_table(b, drop)

    def in_map(i):
        p = i
        for di in drop:
            p = jnp.where(i == di, prev_kept[di], p)
        return (p, 0, 0, 0)

    def last_map(i):
        p = prev_drop[0]
        for idx in range(1, b):
            if prev_drop[idx] != prev_drop[idx - 1]:
                p = jnp.where(i >= idx, prev_drop[idx], p)
        return (p, t // LH - 1, 0, 0)

    return pl.pallas_call(
        functools.partial(_masked_copy_kernel, drop=drop, t=t),
        grid=(b,),
        in_specs=[
            pl.BlockSpec((1, t, c, d), in_map),
            pl.BlockSpec((1, LH, c, d), last_map),
        ],
        out_specs=pl.BlockSpec((1, t, c, d), lambda i: (i, 0, 0, 0)),
        out_shape=jax.ShapeDtypeStruct((b, t, c, d), emb0.dtype),
        compiler_params=pltpu.CompilerParams(
            dimension_semantics=("arbitrary",)),
    )(emb0, emb0)


_drop_indices(16)


def kernel(emb0):
    return _run(emb0, _drop_indices(emb0.shape[0]))
